# baseline (speedup 1.0000x reference)
import functools

import jax
import jax.numpy as jnp
from jax import lax
from jax.experimental import pallas as pl
from jax.experimental.pallas import tpu as pltpu
from jax.experimental.pallas import tpu_sc as plsc

NUM_EMB = 1000000
EMB_DIM = 64
BASE_DIM = 128
BATCH = 16384
FIELDS = 26

_NROWS = BATCH * FIELDS
_NPAIR = _NROWS // 2

_NC, _NS = 2, 16
_NW = _NC * _NS

_IPW = _NROWS // _NW
_C = 512
_STEPS = _IPW // _C
_NSTREAM = _C // 128


def _sc_gather(tablep, idxf):
  mesh = plsc.VectorSubcoreMesh(core_axis_name="c", subcore_axis_name="s")

  @functools.partial(
      pl.kernel,
      mesh=mesh,
      compiler_params=pltpu.CompilerParams(needs_layout_passes=False),
      out_type=jax.ShapeDtypeStruct((_NPAIR, 2 * EMB_DIM), jnp.float32),
      scratch_types=[
          pltpu.VMEM((_C,), jnp.int32),
          pltpu.VMEM((_C,), jnp.int32),
          pltpu.VMEM((_C, 2 * EMB_DIM), jnp.float32),
          pltpu.VMEM((_C // 2, 2 * EMB_DIM), jnp.float32),
          pltpu.SemaphoreType.DMA,
      ],
  )
  def k(tab_hbm, idx_hbm, out_hbm, idx_v, p_v, g_v, s_v, sem):
    wid = lax.axis_index("s") * _NC + lax.axis_index("c")
    base = wid * _IPW

    def step(t, _):
      off = base + t * _C
      pltpu.sync_copy(idx_hbm.at[pl.ds(off, _C)], idx_v)
      for v in range(_C // 16):
        sl = pl.ds(v * 16, 16)
        p_v[sl] = lax.shift_right_logical(idx_v[sl], 1)
      copies = []
      for j in range(_NSTREAM):
        copies.append(
            pltpu.async_copy(
                tab_hbm.at[p_v.at[pl.ds(j * 128, 128)]],
                g_v.at[pl.ds(j * 128, 128)],
                sem))
      for cp in copies:
        cp.wait()

      lanes = lax.iota(jnp.int32, 16)
      srow_rel = lax.shift_right_logical(lanes, 1)
      scol0 = (lanes & 1) * EMB_DIM

      def sel(gi, _):
        c0 = gi * 16
        iv = idx_v[pl.ds(c0, 16)]
        rows = c0 + lanes
        hoff = (iv & 1) * EMB_DIM
        srow = (c0 // 2) + srow_rel
        for d in range(EMB_DIM):
          vals = plsc.load_gather(g_v, [rows, hoff + d])
          plsc.store_scatter(s_v, [srow, scol0 + d], vals)
        return 0

      lax.fori_loop(0, _C // 16, sel, 0)
      pltpu.sync_copy(
          s_v, out_hbm.at[pl.ds(pl.multiple_of(off // 2, _C // 2), _C // 2)])
      return 0

    lax.fori_loop(0, _STEPS, step, 0)

  return k(tablep, idxf)


_BB = 128
_XROWS = _BB * FIELDS // 2


def _mm_body(x_ref, w2_ref, o_ref):
  y = lax.dot_general(
      x_ref[...], w2_ref[...],
      dimension_numbers=(((1,), (0,)), ((), ())),
      preferred_element_type=jnp.float32)
  o_ref[...] = y.reshape(_BB, FIELDS, BASE_DIM)


def _tc_project(staging, W2):
  return pl.pallas_call(
      _mm_body,
      grid=(BATCH // _BB,),
      in_specs=[
          pl.BlockSpec((_XROWS, 2 * EMB_DIM), lambda i: (i, 0)),
          pl.BlockSpec((2 * EMB_DIM, 2 * BASE_DIM), lambda i: (0, 0)),
      ],
      out_specs=pl.BlockSpec((_BB, FIELDS, BASE_DIM), lambda i: (i, 0, 0)),
      out_shape=jax.ShapeDtypeStruct((BATCH, FIELDS, BASE_DIM), jnp.float32),
  )(staging, W2)


def kernel(input, table, W):
  idxf = input.astype(jnp.int32).reshape(_NROWS)
  tablep = table.reshape(NUM_EMB // 2, 2 * EMB_DIM)
  Wt = W.T
  W2 = jnp.zeros((2 * EMB_DIM, 2 * BASE_DIM), jnp.float32)
  W2 = W2.at[:EMB_DIM, :BASE_DIM].set(Wt)
  W2 = W2.at[EMB_DIM:, BASE_DIM:].set(Wt)
  staging = _sc_gather(tablep, idxf)
  return _tc_project(staging, W2)


# speedup vs baseline: 1.9002x; 1.9002x over previous
import functools

import jax
import jax.numpy as jnp
from jax import lax
from jax.experimental import pallas as pl
from jax.experimental.pallas import tpu as pltpu
from jax.experimental.pallas import tpu_sc as plsc

NUM_EMB = 1000000
EMB_DIM = 64
BASE_DIM = 128
BATCH = 16384
FIELDS = 26

_NROWS = BATCH * FIELDS
_IDXW = 128
_IDX_ROWS = _NROWS // _IDXW

_NC, _NS = 2, 16
_NW = _NC * _NS

_IPW = _NROWS // _NW
_C = 512
_NSTREAM = _C // _IDXW
_STEPS = _IPW // _C


def _sc_gather(tpad, idx2d):
  mesh = plsc.VectorSubcoreMesh(core_axis_name="c", subcore_axis_name="s")

  @functools.partial(
      pl.kernel,
      mesh=mesh,
      out_type=jax.ShapeDtypeStruct((_NROWS, 2 * EMB_DIM), jnp.float32),
      scratch_types=[
          pltpu.VMEM((_NSTREAM, _IDXW), jnp.int32),
          pltpu.VMEM((_C, 2 * EMB_DIM), jnp.float32),
          pltpu.SemaphoreType.DMA,
      ],
  )
  def k(tab_hbm, idx_hbm, out_hbm, idx_v, g_v, sem):
    wid = lax.axis_index("s") * _NC + lax.axis_index("c")
    idx_row0 = wid * (_IPW // _IDXW)
    row0 = wid * _IPW

    def step(t, _):
      pltpu.sync_copy(
          idx_hbm.at[pl.ds(idx_row0 + t * _NSTREAM, _NSTREAM)], idx_v)
      copies = []
      for j in range(_NSTREAM):
        copies.append(
            pltpu.async_copy(
                tab_hbm.at[idx_v.at[j]],
                g_v.at[pl.ds(j * _IDXW, _IDXW)],
                sem))
      for cp in copies:
        cp.wait()
      pltpu.sync_copy(
          g_v,
          out_hbm.at[pl.ds(pl.multiple_of(row0 + t * _C, _C), _C)])
      return 0

    lax.fori_loop(0, _STEPS, step, 0)

  return k(tpad, idx2d)


_BB = 128
_XR = _BB * FIELDS


def _mm_body(x_ref, w_ref, o_ref):
  y = lax.dot_general(
      x_ref[...], w_ref[...],
      dimension_numbers=(((1,), (0,)), ((), ())),
      preferred_element_type=jnp.float32)
  o_ref[...] = y.reshape(_BB, FIELDS, BASE_DIM)


def _tc_project(staging, Wz):
  return pl.pallas_call(
      _mm_body,
      grid=(BATCH // _BB,),
      in_specs=[
          pl.BlockSpec((_XR, 2 * EMB_DIM), lambda i: (i, 0)),
          pl.BlockSpec((2 * EMB_DIM, BASE_DIM), lambda i: (0, 0)),
      ],
      out_specs=pl.BlockSpec((_BB, FIELDS, BASE_DIM), lambda i: (i, 0, 0)),
      out_shape=jax.ShapeDtypeStruct((BATCH, FIELDS, BASE_DIM), jnp.float32),
  )(staging, Wz)


def kernel(input, table, W):
  idx2d = input.astype(jnp.int32).reshape(_IDX_ROWS, _IDXW)
  tpad = jnp.pad(table, ((0, 0), (0, EMB_DIM)))
  Wz = jnp.pad(W.T, ((0, EMB_DIM), (0, 0)))
  staging = _sc_gather(tpad, idx2d)
  return _tc_project(staging, Wz)


# speedup vs baseline: 3.0780x; 1.6198x over previous
import functools

import jax
import jax.numpy as jnp
from jax import lax
from jax.experimental import pallas as pl
from jax.experimental.pallas import tpu as pltpu
from jax.experimental.pallas import tpu_sc as plsc

NUM_EMB = 1000000
EMB_DIM = 64
BASE_DIM = 128
BATCH = 16384
FIELDS = 26

_NROWS = BATCH * FIELDS
_IDXW = 128
_IDX_ROWS = _NROWS // _IDXW

_NC, _NS = 2, 16
_NW = _NC * _NS

_IPW = _NROWS // _NW
_C = 512
_NSTREAM = _C // _IDXW
_STEPS = _IPW // _C

_PB = 2048
_PGRID = (NUM_EMB + _PB - 1) // _PB


def _proj_body(xt_ref, w_ref, o_ref):
  o_ref[...] = lax.dot_general(
      xt_ref[...], w_ref[...],
      dimension_numbers=(((0,), (0,)), ((), ())),
      preferred_element_type=jnp.float32)


def _tc_project_table(tableT, Wt):
  return pl.pallas_call(
      _proj_body,
      grid=(_PGRID,),
      in_specs=[
          pl.BlockSpec((EMB_DIM, _PB), lambda i: (0, i)),
          pl.BlockSpec((EMB_DIM, BASE_DIM), lambda i: (0, 0)),
      ],
      out_specs=pl.BlockSpec((_PB, BASE_DIM), lambda i: (i, 0)),
      out_shape=jax.ShapeDtypeStruct((NUM_EMB, BASE_DIM), jnp.float32),
  )(tableT, Wt)


def _sc_gather(proj, idx2d):
  mesh = plsc.VectorSubcoreMesh(core_axis_name="c", subcore_axis_name="s")

  @functools.partial(
      pl.kernel,
      mesh=mesh,
      out_type=jax.ShapeDtypeStruct((_NROWS, BASE_DIM), jnp.float32),
      scratch_types=[
          pltpu.VMEM((_NSTREAM, _IDXW), jnp.int32),
          pltpu.VMEM((_C, BASE_DIM), jnp.float32),
          pltpu.SemaphoreType.DMA,
      ],
  )
  def k(p_hbm, idx_hbm, out_hbm, idx_v, g_v, sem):
    wid = lax.axis_index("s") * _NC + lax.axis_index("c")
    idx_row0 = wid * (_IPW // _IDXW)
    row0 = wid * _IPW

    def step(t, _):
      pltpu.sync_copy(
          idx_hbm.at[pl.ds(idx_row0 + t * _NSTREAM, _NSTREAM)], idx_v)
      copies = []
      for j in range(_NSTREAM):
        copies.append(
            pltpu.async_copy(
                p_hbm.at[idx_v.at[j]],
                g_v.at[pl.ds(j * _IDXW, _IDXW)],
                sem))
      for cp in copies:
        cp.wait()
      pltpu.sync_copy(
          g_v,
          out_hbm.at[pl.ds(pl.multiple_of(row0 + t * _C, _C), _C)])
      return 0

    lax.fori_loop(0, _STEPS, step, 0)

  return k(proj, idx2d)


def kernel(input, table, W):
  idx2d = input.astype(jnp.int32).T.reshape(_IDX_ROWS, _IDXW)
  proj = _tc_project_table(table.T, W.T)
  rows = _sc_gather(proj, idx2d)
  return rows.reshape(FIELDS, BATCH, BASE_DIM).transpose(1, 0, 2)


# speedup vs baseline: 4.5081x; 1.4646x over previous
import functools

import jax
import jax.numpy as jnp
from jax import lax
from jax.experimental import pallas as pl
from jax.experimental.pallas import tpu as pltpu
from jax.experimental.pallas import tpu_sc as plsc

NUM_EMB = 1000000
EMB_DIM = 64
BASE_DIM = 128
BATCH = 16384
FIELDS = 26

_NROWS = BATCH * FIELDS
_IDXW = 128
_IDX_ROWS = _NROWS // _IDXW

_NC, _NS = 2, 16
_NW = _NC * _NS

_IPW = _NROWS // _NW
_C = 512
_NSTREAM = _C // _IDXW
_STEPS = _IPW // _C

_PB = 8192
_PGRID = (NUM_EMB + _PB - 1) // _PB


def _proj_body(xt_ref, w_ref, o_ref):
  o_ref[...] = lax.dot_general(
      xt_ref[...], w_ref[...],
      dimension_numbers=(((0,), (0,)), ((), ())),
      preferred_element_type=jnp.float32)


def _tc_project_table(tableT, Wt):
  return pl.pallas_call(
      _proj_body,
      compiler_params=pltpu.CompilerParams(fuse_transposed_lhs_in_matmul=True),
      grid=(_PGRID,),
      in_specs=[
          pl.BlockSpec((EMB_DIM, _PB), lambda i: (0, i)),
          pl.BlockSpec((EMB_DIM, BASE_DIM), lambda i: (0, 0)),
      ],
      out_specs=pl.BlockSpec((_PB, BASE_DIM), lambda i: (i, 0)),
      out_shape=jax.ShapeDtypeStruct((NUM_EMB, BASE_DIM), jnp.float32),
  )(tableT, Wt)


def _sc_gather(proj, idx2d):
  mesh = plsc.VectorSubcoreMesh(core_axis_name="c", subcore_axis_name="s")

  @functools.partial(
      pl.kernel,
      mesh=mesh,
      out_type=jax.ShapeDtypeStruct((_NROWS, BASE_DIM), jnp.float32),
      scratch_types=[
          pltpu.VMEM((_NSTREAM, _IDXW), jnp.int32),
          pltpu.VMEM((_C, BASE_DIM), jnp.float32),
          pltpu.SemaphoreType.DMA,
      ],
  )
  def k(p_hbm, idx_hbm, out_hbm, idx_v, g_v, sem):
    wid = lax.axis_index("s") * _NC + lax.axis_index("c")
    idx_row0 = wid * (_IPW // _IDXW)
    row0 = wid * _IPW

    def step(t, _):
      pltpu.sync_copy(
          idx_hbm.at[pl.ds(idx_row0 + t * _NSTREAM, _NSTREAM)], idx_v)
      copies = []
      for j in range(_NSTREAM):
        copies.append(
            pltpu.async_copy(
                p_hbm.at[idx_v.at[j]],
                g_v.at[pl.ds(j * _IDXW, _IDXW)],
                sem))
      for cp in copies:
        cp.wait()
      pltpu.sync_copy(
          g_v,
          out_hbm.at[pl.ds(pl.multiple_of(row0 + t * _C, _C), _C)])
      return 0

    lax.fori_loop(0, _STEPS, step, 0)

  return k(proj, idx2d)


def kernel(input, table, W):
  idx2d = input.astype(jnp.int32).T.reshape(_IDX_ROWS, _IDXW)
  proj = _tc_project_table(table.T, W.T)
  rows = _sc_gather(proj, idx2d)
  return rows.reshape(FIELDS, BATCH, BASE_DIM).transpose(1, 0, 2)


# speedup vs baseline: 4.7655x; 1.0571x over previous
import functools

import jax
import jax.numpy as jnp
from jax import lax
from jax.experimental import pallas as pl
from jax.experimental.pallas import tpu as pltpu
from jax.experimental.pallas import tpu_sc as plsc

NUM_EMB = 1000000
EMB_DIM = 64
BASE_DIM = 128
BATCH = 16384
FIELDS = 26

_NROWS = BATCH * FIELDS
_IDXW = 128
_IDX_ROWS = _NROWS // _IDXW

_NC, _NS = 2, 16
_NW = _NC * _NS

_IPW = _NROWS // _NW
_C = 512
_NSTREAM = _C // _IDXW
_STEPS = _IPW // _C

_PB = 16384
_PGRID = (NUM_EMB + _PB - 1) // _PB


def _proj_body(xt_ref, w_ref, o_ref):
  o_ref[...] = lax.dot_general(
      xt_ref[...], w_ref[...],
      dimension_numbers=(((0,), (0,)), ((), ())),
      preferred_element_type=jnp.float32)


def _tc_project_table(tableT, Wt):
  return pl.pallas_call(
      _proj_body,
      compiler_params=pltpu.CompilerParams(fuse_transposed_lhs_in_matmul=True),
      grid=(_PGRID,),
      in_specs=[
          pl.BlockSpec((EMB_DIM, _PB), lambda i: (0, i)),
          pl.BlockSpec((EMB_DIM, BASE_DIM), lambda i: (0, 0)),
      ],
      out_specs=pl.BlockSpec((_PB, BASE_DIM), lambda i: (i, 0)),
      out_shape=jax.ShapeDtypeStruct((NUM_EMB, BASE_DIM), jnp.float32),
  )(tableT, Wt)


def _sc_gather(proj, idx2d):
  mesh = plsc.VectorSubcoreMesh(core_axis_name="c", subcore_axis_name="s")

  @functools.partial(
      pl.kernel,
      mesh=mesh,
      out_type=jax.ShapeDtypeStruct((_NROWS, BASE_DIM), jnp.float32),
      scratch_types=[
          pltpu.VMEM((_NSTREAM, _IDXW), jnp.int32),
          pltpu.VMEM((_C, BASE_DIM), jnp.float32),
          pltpu.SemaphoreType.DMA,
      ],
  )
  def k(p_hbm, idx_hbm, out_hbm, idx_v, g_v, sem):
    wid = lax.axis_index("s") * _NC + lax.axis_index("c")
    idx_row0 = wid * (_IPW // _IDXW)
    row0 = wid * _IPW

    def step(t, _):
      pltpu.sync_copy(
          idx_hbm.at[pl.ds(idx_row0 + t * _NSTREAM, _NSTREAM)], idx_v)
      copies = []
      for j in range(_NSTREAM):
        copies.append(
            pltpu.async_copy(
                p_hbm.at[idx_v.at[j]],
                g_v.at[pl.ds(j * _IDXW, _IDXW)],
                sem))
      for cp in copies:
        cp.wait()
      pltpu.sync_copy(
          g_v,
          out_hbm.at[pl.ds(pl.multiple_of(row0 + t * _C, _C), _C)])
      return 0

    lax.fori_loop(0, _STEPS, step, 0)

  return k(proj, idx2d)


def kernel(input, table, W):
  idx2d = input.astype(jnp.int32).T.reshape(_IDX_ROWS, _IDXW)
  proj = _tc_project_table(table.T, W.T)
  rows = _sc_gather(proj, idx2d)
  return rows.reshape(FIELDS, BATCH, BASE_DIM).transpose(1, 0, 2)


# speedup vs baseline: 5.0127x; 1.0519x over previous
import functools

import jax
import jax.numpy as jnp
from jax import lax
from jax.experimental import pallas as pl
from jax.experimental.pallas import tpu as pltpu
from jax.experimental.pallas import tpu_sc as plsc

NUM_EMB = 1000000
EMB_DIM = 64
BASE_DIM = 128
BATCH = 16384
FIELDS = 26

_NROWS = BATCH * FIELDS
_IDXW = 128
_IDX_ROWS = _NROWS // _IDXW

_NC, _NS = 2, 16
_NW = _NC * _NS

_IPW = _NROWS // _NW
_IDXR_PW = _IPW // _IDXW
_C = 256
_NSTREAM = _C // _IDXW
_STEPS = _IPW // _C

_PB = 16384
_PGRID = (NUM_EMB + _PB - 1) // _PB


def _proj_body(xt_ref, w_ref, o_ref):
  o_ref[...] = lax.dot_general(
      xt_ref[...], w_ref[...],
      dimension_numbers=(((0,), (0,)), ((), ())),
      preferred_element_type=jnp.float32)


def _tc_project_table(tableT, Wt):
  return pl.pallas_call(
      _proj_body,
      compiler_params=pltpu.CompilerParams(fuse_transposed_lhs_in_matmul=True),
      grid=(_PGRID,),
      in_specs=[
          pl.BlockSpec((EMB_DIM, _PB), lambda i: (0, i)),
          pl.BlockSpec((EMB_DIM, BASE_DIM), lambda i: (0, 0)),
      ],
      out_specs=pl.BlockSpec((_PB, BASE_DIM), lambda i: (i, 0)),
      out_shape=jax.ShapeDtypeStruct((NUM_EMB, BASE_DIM), jnp.float32),
  )(tableT, Wt)


def _sc_gather(proj, idx2d):
  mesh = plsc.VectorSubcoreMesh(core_axis_name="c", subcore_axis_name="s")

  @functools.partial(
      pl.kernel,
      mesh=mesh,
      out_type=jax.ShapeDtypeStruct((_NROWS, BASE_DIM), jnp.float32),
      scratch_types=[
          pltpu.VMEM((_IDXR_PW, _IDXW), jnp.int32),
          pltpu.VMEM((_C, BASE_DIM), jnp.float32),
          pltpu.VMEM((_C, BASE_DIM), jnp.float32),
          pltpu.SemaphoreType.DMA,
          pltpu.SemaphoreType.DMA,
          pltpu.SemaphoreType.DMA,
          pltpu.SemaphoreType.DMA,
      ],
  )
  def k(p_hbm, idx_hbm, out_hbm, idx_v, g0, g1, sg0, sg1, sw0, sw1):
    wid = lax.axis_index("s") * _NC + lax.axis_index("c")
    idx_row0 = wid * _IDXR_PW
    row0 = wid * _IPW

    pltpu.sync_copy(idx_hbm.at[pl.ds(idx_row0, _IDXR_PW)], idx_v)

    bufs = (g0, g1)
    gsems = (sg0, sg1)
    wsems = (sw0, sw1)

    def fire(t):
      buf, sem = bufs[t % 2], gsems[t % 2]
      return [
          pltpu.async_copy(
              p_hbm.at[idx_v.at[t * _NSTREAM + j]],
              buf.at[pl.ds(j * _IDXW, _IDXW)],
              sem)
          for j in range(_NSTREAM)
      ]

    pend_g = fire(0)
    pend_w = [None, None]
    for t in range(_STEPS):
      b = t % 2
      if t + 1 < _STEPS:
        nb = (t + 1) % 2
        if pend_w[nb] is not None:
          pend_w[nb].wait()
          pend_w[nb] = None
        next_g = fire(t + 1)
      for cp in pend_g:
        cp.wait()
      if t + 1 < _STEPS:
        pend_g = next_g
      desc = pltpu.make_async_copy(
          bufs[b], out_hbm.at[pl.ds(row0 + t * _C, _C)], wsems[b])
      desc.start()
      pend_w[b] = desc
    for d in pend_w:
      if d is not None:
        d.wait()

  return k(proj, idx2d)


def kernel(input, table, W):
  idx2d = input.astype(jnp.int32).T.reshape(_IDX_ROWS, _IDXW)
  proj = _tc_project_table(table.T, W.T)
  rows = _sc_gather(proj, idx2d)
  return rows.reshape(FIELDS, BATCH, BASE_DIM).transpose(1, 0, 2)
